# baseline (speedup 1.0000x reference)
import jax
import jax.numpy as jnp
import numpy as np
from jax.experimental import pallas as pl
from jax.experimental.pallas import tpu as pltpu

N = 10000
D = 128
H1 = 64
H2 = 32

BM = 200
NSTRIP = N // BM
RB = 400
CB = 2560
NROW = N // RB
NPAN = -(-N // CB)
MUPAD = NPAN * CB
MAXW = 2


def _build_schedule():
    rth = [-(-min(CB * (p + 1), N) // BM) - 1 for p in range(NPAN)]
    ready = {}
    for i in range(NROW):
        for p in range(NPAN):
            ready.setdefault(max(2 * i + 1, rth[p]), []).append((i, p))
    steps = []
    for r in range(NSTRIP):
        steps.append([0, r, -1, -1, 0])
    pending = []
    for r in range(NSTRIP):
        steps.append([1, r, -1, -1, r])
        pending.extend(ready.get(r, []))
        for _ in range(MAXW):
            if pending and r < NSTRIP - 1:
                i, p = pending.pop(0)
                steps.append([2, r, i, p, r])
    while pending:
        i, p = pending.pop(0)
        steps.append([2, NSTRIP - 1, i, p, NSTRIP - 1])
    ni, npp = steps[-1][2], steps[-1][3]
    for s in reversed(steps):
        if s[0] == 2:
            ni, npp = s[2], s[3]
        else:
            s[2], s[3] = ni, npp
    return np.asarray(steps, dtype=np.int32)

_SCHED = _build_schedule()
_NSTEPS = _SCHED.shape[0]


def _k1_small(x_ref, w1_ref, wa1_ref, wa2_ref, wa3_ref,
              xw1_ref, mua_ref, logvara_ref):
    x = x_ref[...]
    xw1_ref[...] = jnp.dot(x, w1_ref[...], preferred_element_type=jnp.float32)
    ha1 = jnp.tanh(jax.lax.dot_general(
        x, wa1_ref[...], (((0,), (0,)), ((), ())),
        preferred_element_type=jnp.float32))
    mua_ref[...] = jnp.dot(ha1, wa2_ref[...], preferred_element_type=jnp.float32)
    logvara_ref[...] = jnp.dot(ha1, wa3_ref[...], preferred_element_type=jnp.float32)


def _k2_fused(sched_ref, adj_ref, xw1_ref, w23_ref, mua_ref,
              mu_ref, logvar_ref, feat_ref, adjrec_ref,
              hw_ref, muf_ref):
    s = pl.program_id(0)
    kind = sched_ref[s, 0]
    r = sched_ref[s, 1]

    @pl.when(s == 0)
    def _zero_pad():
        muf_ref[pl.ds(N, MUPAD - N), :] = jnp.zeros((MUPAD - N, H2),
                                                    jnp.float32)

    @pl.when(kind == 0)
    def _pass1():
        h1 = jnp.maximum(
            jnp.dot(adj_ref[...], xw1_ref[...],
                    preferred_element_type=jnp.float32), 0.0)
        hw_ref[pl.ds(r * BM, BM), :] = jnp.dot(
            h1, w23_ref[...], preferred_element_type=jnp.float32)

    @pl.when(kind == 1)
    def _pass2():
        ml = jnp.dot(adj_ref[...], hw_ref[...],
                     preferred_element_type=jnp.float32)
        mu = ml[:, :H2]
        mu_ref[...] = mu
        logvar_ref[...] = ml[:, H2:]
        muf_ref[pl.ds(r * BM, BM), :] = mu
        feat_ref[...] = jax.lax.dot_general(
            mu, mua_ref[...], (((1,), (1,)), ((), ())),
            preferred_element_type=jnp.float32)

    @pl.when(kind == 2)
    def _decoder():
        i = sched_ref[s, 2]
        p = sched_ref[s, 3]
        zi = muf_ref[pl.ds(i * RB, RB), :]
        zj = muf_ref[pl.ds(p * CB, CB), :]
        adjrec_ref[...] = jax.lax.dot_general(
            zi, zj, (((1,), (1,)), ((), ())),
            preferred_element_type=jnp.float32)


def kernel(x, adj, W1, W2, W3, Wa1, Wa2, Wa3):
    f32 = jnp.float32

    xw1, mu_a, logvar_a = pl.pallas_call(
        _k1_small,
        out_shape=(
            jax.ShapeDtypeStruct((N, H1), f32),
            jax.ShapeDtypeStruct((D, H2), f32),
            jax.ShapeDtypeStruct((D, H2), f32),
        ),
    )(x, W1, Wa1, Wa2, Wa3)

    w23 = jnp.concatenate([W2, W3], axis=1)

    grid_spec = pltpu.PrefetchScalarGridSpec(
        num_scalar_prefetch=1,
        grid=(_NSTEPS,),
        in_specs=[
            pl.BlockSpec((BM, N), lambda s, sr: (sr[s, 1], 0)),
            pl.BlockSpec((N, H1), lambda s, sr: (0, 0)),
            pl.BlockSpec((H1, 2 * H2), lambda s, sr: (0, 0)),
            pl.BlockSpec((D, H2), lambda s, sr: (0, 0)),
        ],
        out_specs=(
            pl.BlockSpec((BM, H2), lambda s, sr: (sr[s, 4], 0)),
            pl.BlockSpec((BM, H2), lambda s, sr: (sr[s, 4], 0)),
            pl.BlockSpec((BM, D), lambda s, sr: (sr[s, 4], 0)),
            pl.BlockSpec((RB, CB), lambda s, sr: (sr[s, 2], sr[s, 3])),
        ),
        scratch_shapes=[
            pltpu.VMEM((N, 2 * H2), f32),
            pltpu.VMEM((MUPAD, H2), f32),
        ],
    )

    mu, logvar, features, adj_rec = pl.pallas_call(
        _k2_fused,
        grid_spec=grid_spec,
        out_shape=(
            jax.ShapeDtypeStruct((N, H2), f32),
            jax.ShapeDtypeStruct((N, H2), f32),
            jax.ShapeDtypeStruct((N, D), f32),
            jax.ShapeDtypeStruct((N, N), f32),
        ),
        compiler_params=pltpu.CompilerParams(
            dimension_semantics=("arbitrary",)),
    )(jnp.asarray(_SCHED), adj, xw1, w23, mu_a)

    return (adj_rec, features, mu, logvar, mu_a, logvar_a)


# speedup vs baseline: 1.3019x; 1.3019x over previous
import jax
import jax.numpy as jnp
from jax.experimental import pallas as pl
from jax.experimental.pallas import tpu as pltpu

N = 10000
D = 128
H1 = 64
H2 = 32

BM = 200
NSTRIP = N // BM


def _k1_small(x_ref, w1_ref, wa1_ref, wa2_ref, wa3_ref,
              xw1_ref, mua_ref, logvara_ref):
    x = x_ref[...]
    xw1_ref[...] = jnp.dot(x, w1_ref[...], preferred_element_type=jnp.float32)
    ha1 = jnp.tanh(jax.lax.dot_general(
        x, wa1_ref[...], (((0,), (0,)), ((), ())),
        preferred_element_type=jnp.float32))
    mua_ref[...] = jnp.dot(ha1, wa2_ref[...], preferred_element_type=jnp.float32)
    logvara_ref[...] = jnp.dot(ha1, wa3_ref[...], preferred_element_type=jnp.float32)


def _k2_fused(adj_ref, xw1_ref, w23_ref, mua_ref,
              mu_ref, logvar_ref, feat_ref, adjrec_ref,
              hw_ref, muf_ref):
    s = pl.program_id(0)
    r = jax.lax.rem(s, NSTRIP)

    @pl.when(s < NSTRIP)
    def _phase0():
        h1 = jnp.maximum(
            jnp.dot(adj_ref[...], xw1_ref[...],
                    preferred_element_type=jnp.float32), 0.0)
        hw_ref[pl.ds(r * BM, BM), :] = jnp.dot(
            h1, w23_ref[...], preferred_element_type=jnp.float32)

    @pl.when(jnp.logical_and(s >= NSTRIP, s < 2 * NSTRIP))
    def _phase1():
        ml = jnp.dot(adj_ref[...], hw_ref[...],
                     preferred_element_type=jnp.float32)
        mu = ml[:, :H2]
        mu_ref[...] = mu
        logvar_ref[...] = ml[:, H2:]
        muf_ref[pl.ds(r * BM, BM), :] = mu
        feat_ref[...] = jax.lax.dot_general(
            mu, mua_ref[...], (((1,), (1,)), ((), ())),
            preferred_element_type=jnp.float32)

    @pl.when(s >= 2 * NSTRIP)
    def _phase2():
        zi = muf_ref[pl.ds(r * BM, BM), :]
        adjrec_ref[...] = jax.lax.dot_general(
            zi, muf_ref[...], (((1,), (1,)), ((), ())),
            preferred_element_type=jnp.float32)


def kernel(x, adj, W1, W2, W3, Wa1, Wa2, Wa3):
    f32 = jnp.float32

    xw1, mu_a, logvar_a = pl.pallas_call(
        _k1_small,
        out_shape=(
            jax.ShapeDtypeStruct((N, H1), f32),
            jax.ShapeDtypeStruct((D, H2), f32),
            jax.ShapeDtypeStruct((D, H2), f32),
        ),
    )(x, W1, Wa1, Wa2, Wa3)

    w23 = jnp.concatenate([W2, W3], axis=1)

    last = NSTRIP - 1
    mu, logvar, features, adj_rec = pl.pallas_call(
        _k2_fused,
        grid=(3 * NSTRIP,),
        in_specs=[
            pl.BlockSpec((BM, N),
                         lambda s: (jnp.where(s >= 2 * NSTRIP, last,
                                              jax.lax.rem(s, NSTRIP)), 0)),
            pl.BlockSpec((N, H1), lambda s: (0, 0)),
            pl.BlockSpec((H1, 2 * H2), lambda s: (0, 0)),
            pl.BlockSpec((D, H2), lambda s: (0, 0)),
        ],
        out_specs=(
            pl.BlockSpec((BM, H2),
                         lambda s: (jnp.clip(s - NSTRIP, 0, last), 0)),
            pl.BlockSpec((BM, H2),
                         lambda s: (jnp.clip(s - NSTRIP, 0, last), 0)),
            pl.BlockSpec((BM, D),
                         lambda s: (jnp.clip(s - NSTRIP, 0, last), 0)),
            pl.BlockSpec((BM, N),
                         lambda s: (jnp.clip(s - 2 * NSTRIP, 0, last), 0)),
        ),
        out_shape=(
            jax.ShapeDtypeStruct((N, H2), f32),
            jax.ShapeDtypeStruct((N, H2), f32),
            jax.ShapeDtypeStruct((N, D), f32),
            jax.ShapeDtypeStruct((N, N), f32),
        ),
        scratch_shapes=[
            pltpu.VMEM((N, 2 * H2), f32),
            pltpu.VMEM((N, H2), f32),
        ],
        compiler_params=pltpu.CompilerParams(
            dimension_semantics=("arbitrary",)),
    )(adj, xw1, w23, mu_a)

    return (adj_rec, features, mu, logvar, mu_a, logvar_a)


# speedup vs baseline: 1.3155x; 1.0104x over previous
import jax
import jax.numpy as jnp
from jax.experimental import pallas as pl
from jax.experimental.pallas import tpu as pltpu

N = 10000
D = 128
H1 = 64
H2 = 32

BM = 200
NSTRIP = N // BM


def _fused(adj_ref, x_ref, w1_ref, wa1_ref, wa2_ref, wa3_ref, w23_ref,
           mu_ref, logvar_ref, feat_ref, adjrec_ref, mua_ref, logvara_ref,
           hw_ref, muf_ref, xw1_ref):
    s = pl.program_id(0)
    r = jax.lax.rem(s, NSTRIP)

    @pl.when(s == 0)
    def _thin_dense():
        x = x_ref[...]
        xw1_ref[...] = jnp.dot(x, w1_ref[...],
                               preferred_element_type=jnp.float32)
        ha1 = jnp.tanh(jax.lax.dot_general(
            x, wa1_ref[...], (((0,), (0,)), ((), ())),
            preferred_element_type=jnp.float32))
        mua_ref[...] = jnp.dot(ha1, wa2_ref[...],
                               preferred_element_type=jnp.float32)
        logvara_ref[...] = jnp.dot(ha1, wa3_ref[...],
                                   preferred_element_type=jnp.float32)

    @pl.when(s < NSTRIP)
    def _phase0():
        h1 = jnp.maximum(
            jnp.dot(adj_ref[...], xw1_ref[...],
                    preferred_element_type=jnp.float32), 0.0)
        hw_ref[pl.ds(r * BM, BM), :] = jnp.dot(
            h1, w23_ref[...], preferred_element_type=jnp.float32)

    @pl.when(jnp.logical_and(s >= NSTRIP, s < 2 * NSTRIP))
    def _phase1():
        ml = jnp.dot(adj_ref[...], hw_ref[...],
                     preferred_element_type=jnp.float32)
        mu = ml[:, :H2]
        mu_ref[...] = mu
        logvar_ref[...] = ml[:, H2:]
        muf_ref[pl.ds(r * BM, BM), :] = mu
        feat_ref[...] = jax.lax.dot_general(
            mu, mua_ref[...], (((1,), (1,)), ((), ())),
            preferred_element_type=jnp.float32)

    @pl.when(s >= 2 * NSTRIP)
    def _phase2():
        zi = muf_ref[pl.ds(r * BM, BM), :]
        adjrec_ref[...] = jax.lax.dot_general(
            zi, muf_ref[...], (((1,), (1,)), ((), ())),
            preferred_element_type=jnp.float32)


def kernel(x, adj, W1, W2, W3, Wa1, Wa2, Wa3):
    f32 = jnp.float32
    w23 = jnp.concatenate([W2, W3], axis=1)

    last = NSTRIP - 1
    mu, logvar, features, adj_rec, mu_a, logvar_a = pl.pallas_call(
        _fused,
        grid=(3 * NSTRIP,),
        in_specs=[
            pl.BlockSpec((BM, N),
                         lambda s: (jnp.where(s >= 2 * NSTRIP, last,
                                              jax.lax.rem(s, NSTRIP)), 0)),
            pl.BlockSpec((N, D), lambda s: (0, 0)),
            pl.BlockSpec((D, H1), lambda s: (0, 0)),
            pl.BlockSpec((N, H1), lambda s: (0, 0)),
            pl.BlockSpec((H1, H2), lambda s: (0, 0)),
            pl.BlockSpec((H1, H2), lambda s: (0, 0)),
            pl.BlockSpec((H1, 2 * H2), lambda s: (0, 0)),
        ],
        out_specs=(
            pl.BlockSpec((BM, H2),
                         lambda s: (jnp.clip(s - NSTRIP, 0, last), 0)),
            pl.BlockSpec((BM, H2),
                         lambda s: (jnp.clip(s - NSTRIP, 0, last), 0)),
            pl.BlockSpec((BM, D),
                         lambda s: (jnp.clip(s - NSTRIP, 0, last), 0)),
            pl.BlockSpec((BM, N),
                         lambda s: (jnp.clip(s - 2 * NSTRIP, 0, last), 0)),
            pl.BlockSpec((D, H2), lambda s: (0, 0)),
            pl.BlockSpec((D, H2), lambda s: (0, 0)),
        ),
        out_shape=(
            jax.ShapeDtypeStruct((N, H2), f32),
            jax.ShapeDtypeStruct((N, H2), f32),
            jax.ShapeDtypeStruct((N, D), f32),
            jax.ShapeDtypeStruct((N, N), f32),
            jax.ShapeDtypeStruct((D, H2), f32),
            jax.ShapeDtypeStruct((D, H2), f32),
        ),
        scratch_shapes=[
            pltpu.VMEM((N, 2 * H2), f32),
            pltpu.VMEM((N, H2), f32),
            pltpu.VMEM((N, H1), f32),
        ],
        compiler_params=pltpu.CompilerParams(
            dimension_semantics=("arbitrary",)),
    )(adj, x, W1, Wa1, Wa2, Wa3, w23)

    return (adj_rec, features, mu, logvar, mu_a, logvar_a)
